# baseline (speedup 1.0000x reference)
import functools

import numpy as np
import jax
import jax.numpy as jnp
from jax import lax
from jax.experimental import pallas as pl
from jax.experimental.pallas import tpu as pltpu
from jax.experimental.pallas import tpu_sc as plsc

_N = 262144
_B = 1024
_NODE_DIM = 20
_X_DIM = 3

_start = np.arccos(np.float32(0.95))
_end = np.arccos(np.float32(0.02))
_ang = np.float32(_start + np.float32(1.0) * (_end - _start))
_ALPHA = np.float32(np.cos(_ang))
_SIGMA = np.float32(np.sin(_ang))
_L = np.float32(np.log(np.float32(1.0) / _SIGMA))
_Q = np.float32(_SIGMA * _SIGMA)
_A2 = np.float32(_ALPHA * _ALPHA)
_WH = np.float32(np.float32(20.0) * (_L + np.float32(0.5) * _Q - np.float32(0.5)))

_TN = 32768
_G = _N // _TN


def _sq_body(h_ref, x_ref, o_ref):
    hb = h_ref[...]
    xb = x_ref[...]
    s = jnp.sum(hb * hb, axis=0) + jnp.sum(xb * xb, axis=0)
    o_ref[...] = s.reshape(1, 1, _TN)


def _node_sq(h, x):
    ht = h.T
    xt = x.T
    out = pl.pallas_call(
        _sq_body,
        grid=(_G,),
        in_specs=[
            pl.BlockSpec((_NODE_DIM, _TN), lambda i: (0, i)),
            pl.BlockSpec((_X_DIM, _TN), lambda i: (0, i)),
        ],
        out_specs=pl.BlockSpec((1, 1, _TN), lambda i: (i, 0, 0)),
        out_shape=jax.ShapeDtypeStruct((_G, 1, _TN), jnp.float32),
    )(ht, xt)
    return out


_NC = 2
_NS = 16
_NW = _NC * _NS
_CHUNK = _N // _NW
_NG = _CHUNK // 16


@functools.cache
def _build_seg_kernel():
  mesh = plsc.VectorSubcoreMesh(
      core_axis_name="c", subcore_axis_name="s",
      num_cores=_NC, num_subcores=_NS)

  @functools.partial(
      pl.kernel,
      out_type=(jax.ShapeDtypeStruct((_NC, _B), jnp.float32),
                jax.ShapeDtypeStruct((_NC, _B), jnp.int32)),
      mesh=mesh,
      scratch_types=[
          pltpu.VMEM((_CHUNK,), jnp.float32),
          pltpu.VMEM((_CHUNK + 16,), jnp.int32),
          pltpu.VMEM((_B,), jnp.float32),
          pltpu.VMEM((_B,), jnp.int32),
          pltpu.VMEM_SHARED((_NS, _B), jnp.float32),
          pltpu.VMEM_SHARED((_NS, _B), jnp.int32),
          pltpu.VMEM((_NS, 128), jnp.float32),
          pltpu.VMEM((_NS, 128), jnp.int32),
          pltpu.VMEM((128,), jnp.float32),
          pltpu.VMEM((128,), jnp.int32),
      ],
      compiler_params=pltpu.CompilerParams(needs_layout_passes=False),
  )
  def _seg_kernel(s_hbm, g_hbm, outs_hbm, outc_hbm,
                  sbuf, gbuf, Sloc, Cloc, Ssh, Csh, Stmp, Ctmp, obuf, cbuf):
    cid = lax.axis_index("c")
    sid = lax.axis_index("s")
    base = (cid * _NS + sid) * _CHUNK
    pltpu.sync_copy(s_hbm.at[pl.ds(base, _CHUNK)], sbuf)
    pltpu.sync_copy(g_hbm.at[pl.ds(base, _CHUNK)], gbuf.at[pl.ds(0, _CHUNK)])
    gbuf[pl.ds(_CHUNK, 16)] = jnp.full((16,), -1, jnp.int32)

    zf = jnp.zeros((16,), jnp.float32)
    zi = jnp.zeros((16,), jnp.int32)

    @plsc.parallel_loop(0, _B // 16, unroll=4)
    def _zero(j):
      Sloc[pl.ds(j * 16, 16)] = zf
      Cloc[pl.ds(j * 16, 16)] = zi

    iota16 = lax.iota(jnp.int32, 16)
    pos = iota16 + 1
    lane15 = iota16 == 15

    @plsc.parallel_loop(0, _NG, unroll=8)
    def _accum(i):
      v = sbuf[pl.ds(i * 16, 16)]
      g = gbuf[pl.ds(i * 16, 16)]
      gn = gbuf[pl.ds(i * 16 + 1, 16)]
      c = plsc.cumsum(v)
      b = g != gn
      madd = b | lane15
      msub = b & (~lane15)
      plsc.addupdate_scatter(Sloc, [g], c, mask=madd)
      plsc.addupdate_scatter(Sloc, [gn], -c, mask=msub)
      plsc.addupdate_scatter(Cloc, [g], pos, mask=madd)
      plsc.addupdate_scatter(Cloc, [gn], -pos, mask=msub)

    pltpu.sync_copy(Sloc, Ssh.at[sid])
    pltpu.sync_copy(Cloc, Csh.at[sid])
    plsc.subcore_barrier()

    @pl.when(sid < _B // 128)
    def _epilogue():
      col = sid * 128
      pltpu.sync_copy(Ssh.at[:, pl.ds(col, 128)], Stmp)
      pltpu.sync_copy(Csh.at[:, pl.ds(col, 128)], Ctmp)

      for k in range(128 // 16):
        acc_s = Stmp[0, pl.ds(k * 16, 16)]
        acc_c = Ctmp[0, pl.ds(k * 16, 16)]
        for r in range(1, _NS):
          acc_s = acc_s + Stmp[r, pl.ds(k * 16, 16)]
          acc_c = acc_c + Ctmp[r, pl.ds(k * 16, 16)]
        obuf[pl.ds(k * 16, 16)] = acc_s
        cbuf[pl.ds(k * 16, 16)] = acc_c

      pltpu.sync_copy(obuf, outs_hbm.at[cid, pl.ds(col, 128)])
      pltpu.sync_copy(cbuf, outc_hbm.at[cid, pl.ds(col, 128)])

  return _seg_kernel


def _comb_body(sp_ref, cp_ref, o_ref):
    acc_s = sp_ref[0, :] + sp_ref[1, :]
    cf = (cp_ref[0, :] + cp_ref[1, :]).astype(jnp.float32)
    d = jnp.float32(3.0) * (cf - jnp.float32(1.0))
    wx = d * _L + jnp.float32(0.5) * (d * _Q) - jnp.float32(0.5) * d
    o_ref[...] = cf * _WH + cf * wx + jnp.float32(0.5) * (_A2 * acc_s)


def _combine(sp, cp):
    return pl.pallas_call(
        _comb_body,
        out_shape=jax.ShapeDtypeStruct((_B,), jnp.float32),
    )(sp, cp)


def kernel(h, x, graph_indices):
    s = _node_sq(h, x).reshape(_N)
    sp, cp = _build_seg_kernel()(s, graph_indices)
    return _combine(sp, cp)


# speedup vs baseline: 1.0306x; 1.0306x over previous
import functools

import numpy as np
import jax
import jax.numpy as jnp
from jax import lax
from jax.experimental import pallas as pl
from jax.experimental.pallas import tpu as pltpu
from jax.experimental.pallas import tpu_sc as plsc

_N = 262144
_B = 1024
_NODE_DIM = 20
_X_DIM = 3

_start = np.arccos(np.float32(0.95))
_end = np.arccos(np.float32(0.02))
_ang = np.float32(_start + np.float32(1.0) * (_end - _start))
_ALPHA = np.float32(np.cos(_ang))
_SIGMA = np.float32(np.sin(_ang))
_L = np.float32(np.log(np.float32(1.0) / _SIGMA))
_Q = np.float32(_SIGMA * _SIGMA)
_A2 = np.float32(_ALPHA * _ALPHA)
_WH = np.float32(np.float32(20.0) * (_L + np.float32(0.5) * _Q - np.float32(0.5)))

_TN = 32768
_G = _N // _TN


def _sq_body(h_ref, x_ref, o_ref):
    hb = h_ref[...]
    xb = x_ref[...]
    s = jnp.sum(hb * hb, axis=0) + jnp.sum(xb * xb, axis=0)
    o_ref[...] = s.reshape(1, 1, _TN)


def _node_sq(h, x):
    ht = h.T
    xt = x.T
    out = pl.pallas_call(
        _sq_body,
        grid=(_G,),
        in_specs=[
            pl.BlockSpec((_NODE_DIM, _TN), lambda i: (0, i)),
            pl.BlockSpec((_X_DIM, _TN), lambda i: (0, i)),
        ],
        out_specs=pl.BlockSpec((1, 1, _TN), lambda i: (i, 0, 0)),
        out_shape=jax.ShapeDtypeStruct((_G, 1, _TN), jnp.float32),
    )(ht, xt)
    return out


_NS = 16
_CHUNK = _N // _NS
_NG = _CHUNK // 16
_SLICE = _B // _NS

@functools.cache
def _build_seg_kernel():
  mesh = plsc.VectorSubcoreMesh(
      core_axis_name="c", subcore_axis_name="s", num_cores=1, num_subcores=_NS)

  @functools.partial(
      pl.kernel,
      out_type=jax.ShapeDtypeStruct((_B,), jnp.float32),
      mesh=mesh,
      scratch_types=[
          pltpu.VMEM((_CHUNK,), jnp.float32),
          pltpu.VMEM((_CHUNK + 16,), jnp.int32),
          pltpu.VMEM((_B,), jnp.float32),
          pltpu.VMEM((_B,), jnp.int32),
          pltpu.VMEM_SHARED((_NS, _B), jnp.float32),
          pltpu.VMEM_SHARED((_NS, _B), jnp.int32),
          pltpu.VMEM((_NS, 128), jnp.float32),
          pltpu.VMEM((_NS, 128), jnp.int32),
          pltpu.VMEM((128,), jnp.float32),
          pltpu.SemaphoreType.DMA,
          pltpu.SemaphoreType.DMA,
      ],
      compiler_params=pltpu.CompilerParams(needs_layout_passes=False),
  )
  def _seg_kernel(s_hbm, g_hbm, out_hbm,
                  sbuf, gbuf, Sloc, Cloc, Ssh, Csh, Stmp, Ctmp, obuf,
                  sem_a, sem_b):
    sid = lax.axis_index("s")
    base = sid * _CHUNK
    half = _CHUNK // 2
    cp_sa = pltpu.async_copy(
        s_hbm.at[pl.ds(base, half)], sbuf.at[pl.ds(0, half)], sem_a)
    cp_ga = pltpu.async_copy(
        g_hbm.at[pl.ds(base, half)], gbuf.at[pl.ds(0, half)], sem_a)
    cp_sb = pltpu.async_copy(
        s_hbm.at[pl.ds(base + half, half)], sbuf.at[pl.ds(half, half)], sem_b)
    cp_gb = pltpu.async_copy(
        g_hbm.at[pl.ds(base + half, half)], gbuf.at[pl.ds(half, half)], sem_b)
    gbuf[pl.ds(_CHUNK, 16)] = jnp.full((16,), -1, jnp.int32)

    zf = jnp.zeros((16,), jnp.float32)
    zi = jnp.zeros((16,), jnp.int32)

    @plsc.parallel_loop(0, _B // 16, unroll=4)
    def _zero(j):
      Sloc[pl.ds(j * 16, 16)] = zf
      Cloc[pl.ds(j * 16, 16)] = zi

    iota16 = lax.iota(jnp.int32, 16)
    pos = iota16 + 1
    lane15 = iota16 == 15

    def _accum(i):
      v = sbuf[pl.ds(i * 16, 16)]
      g = gbuf[pl.ds(i * 16, 16)]
      gn = gbuf[pl.ds(i * 16 + 1, 16)]
      c = plsc.cumsum(v)
      b = g != gn
      madd = b | lane15
      msub = b & (~lane15)
      plsc.addupdate_scatter(Sloc, [g], c, mask=madd)
      plsc.addupdate_scatter(Sloc, [gn], -c, mask=msub)
      plsc.addupdate_scatter(Cloc, [g], pos, mask=madd)
      plsc.addupdate_scatter(Cloc, [gn], -pos, mask=msub)

    _split = 504
    cp_sa.wait()
    cp_ga.wait()
    plsc.parallel_loop(0, _split, unroll=8)(_accum)
    cp_sb.wait()
    cp_gb.wait()
    plsc.parallel_loop(_split, _NG, unroll=8)(_accum)

    pltpu.sync_copy(Sloc, Ssh.at[sid])
    pltpu.sync_copy(Cloc, Csh.at[sid])
    plsc.subcore_barrier()

    @pl.when(sid < _B // 128)
    def _epilogue():
      col = sid * 128
      pltpu.sync_copy(Ssh.at[:, pl.ds(col, 128)], Stmp)
      pltpu.sync_copy(Csh.at[:, pl.ds(col, 128)], Ctmp)

      for k in range(128 // 16):
        acc_s = Stmp[0, pl.ds(k * 16, 16)]
        acc_c = Ctmp[0, pl.ds(k * 16, 16)]
        for r in range(1, _NS):
          acc_s = acc_s + Stmp[r, pl.ds(k * 16, 16)]
          acc_c = acc_c + Ctmp[r, pl.ds(k * 16, 16)]
        cf = acc_c.astype(jnp.float32)
        d = jnp.float32(3.0) * (cf - jnp.float32(1.0))
        wx = d * _L + jnp.float32(0.5) * (d * _Q) - jnp.float32(0.5) * d
        outv = cf * _WH + cf * wx + jnp.float32(0.5) * (_A2 * acc_s)
        obuf[pl.ds(k * 16, 16)] = outv

      pltpu.sync_copy(obuf, out_hbm.at[pl.ds(col, 128)])

  return _seg_kernel


def kernel(h, x, graph_indices):
    s = _node_sq(h, x).reshape(_N)
    return _build_seg_kernel()(s, graph_indices)


# speedup vs baseline: 1.0665x; 1.0348x over previous
import functools

import numpy as np
import jax
import jax.numpy as jnp
from jax import lax
from jax.experimental import pallas as pl
from jax.experimental.pallas import tpu as pltpu
from jax.experimental.pallas import tpu_sc as plsc

_N = 262144
_B = 1024
_NODE_DIM = 20
_X_DIM = 3

_start = np.arccos(np.float32(0.95))
_end = np.arccos(np.float32(0.02))
_ang = np.float32(_start + np.float32(1.0) * (_end - _start))
_ALPHA = np.float32(np.cos(_ang))
_SIGMA = np.float32(np.sin(_ang))
_L = np.float32(np.log(np.float32(1.0) / _SIGMA))
_Q = np.float32(_SIGMA * _SIGMA)
_A2 = np.float32(_ALPHA * _ALPHA)
_WH = np.float32(np.float32(20.0) * (_L + np.float32(0.5) * _Q - np.float32(0.5)))

_TN = 65536
_G = _N // _TN


def _sq_body(h_ref, x_ref, o_ref):
    hb = h_ref[...]
    xb = x_ref[...]
    s = jnp.sum(hb * hb, axis=0) + jnp.sum(xb * xb, axis=0)
    o_ref[...] = s.reshape(1, 1, _TN)


def _node_sq(h, x):
    ht = h.T
    xt = x.T
    out = pl.pallas_call(
        _sq_body,
        grid=(_G,),
        in_specs=[
            pl.BlockSpec((_NODE_DIM, _TN), lambda i: (0, i)),
            pl.BlockSpec((_X_DIM, _TN), lambda i: (0, i)),
        ],
        out_specs=pl.BlockSpec((1, 1, _TN), lambda i: (i, 0, 0)),
        out_shape=jax.ShapeDtypeStruct((_G, 1, _TN), jnp.float32),
    )(ht, xt)
    return out


_NS = 16
_CHUNK = _N // _NS
_NG = _CHUNK // 16
_SLICE = _B // _NS

@functools.cache
def _build_seg_kernel():
  mesh = plsc.VectorSubcoreMesh(
      core_axis_name="c", subcore_axis_name="s", num_cores=1, num_subcores=_NS)

  @functools.partial(
      pl.kernel,
      out_type=jax.ShapeDtypeStruct((_B,), jnp.float32),
      mesh=mesh,
      scratch_types=[
          pltpu.VMEM((_CHUNK,), jnp.float32),
          pltpu.VMEM((_CHUNK + 16,), jnp.int32),
          pltpu.VMEM((_B,), jnp.float32),
          pltpu.VMEM((_B,), jnp.int32),
          pltpu.VMEM_SHARED((_NS, _B), jnp.float32),
          pltpu.VMEM_SHARED((_NS, _B), jnp.int32),
          pltpu.VMEM((_NS, 128), jnp.float32),
          pltpu.VMEM((_NS, 128), jnp.int32),
          pltpu.VMEM((128,), jnp.float32),
          pltpu.SemaphoreType.DMA,
          pltpu.SemaphoreType.DMA,
      ],
      compiler_params=pltpu.CompilerParams(needs_layout_passes=False),
  )
  def _seg_kernel(s_hbm, g_hbm, out_hbm,
                  sbuf, gbuf, Sloc, Cloc, Ssh, Csh, Stmp, Ctmp, obuf,
                  sem_a, sem_b):
    sid = lax.axis_index("s")
    base = sid * _CHUNK
    half = _CHUNK // 2
    cp_sa = pltpu.async_copy(
        s_hbm.at[pl.ds(base, half)], sbuf.at[pl.ds(0, half)], sem_a)
    cp_ga = pltpu.async_copy(
        g_hbm.at[pl.ds(base, half)], gbuf.at[pl.ds(0, half)], sem_a)
    cp_sb = pltpu.async_copy(
        s_hbm.at[pl.ds(base + half, half)], sbuf.at[pl.ds(half, half)], sem_b)
    cp_gb = pltpu.async_copy(
        g_hbm.at[pl.ds(base + half, half)], gbuf.at[pl.ds(half, half)], sem_b)
    gbuf[pl.ds(_CHUNK, 16)] = jnp.full((16,), -1, jnp.int32)

    zf = jnp.zeros((16,), jnp.float32)
    zi = jnp.zeros((16,), jnp.int32)

    @plsc.parallel_loop(0, _B // 16, unroll=4)
    def _zero(j):
      Sloc[pl.ds(j * 16, 16)] = zf
      Cloc[pl.ds(j * 16, 16)] = zi

    iota16 = lax.iota(jnp.int32, 16)
    pos = iota16 + 1
    lane15 = iota16 == 15

    def _accum(i):
      v = sbuf[pl.ds(i * 16, 16)]
      g = gbuf[pl.ds(i * 16, 16)]
      gn = gbuf[pl.ds(i * 16 + 1, 16)]
      c = plsc.cumsum(v)
      b = g != gn
      madd = b | lane15
      msub = b & (~lane15)
      plsc.addupdate_scatter(Sloc, [g], c, mask=madd)
      plsc.addupdate_scatter(Sloc, [gn], -c, mask=msub)
      plsc.addupdate_scatter(Cloc, [g], pos, mask=madd)
      plsc.addupdate_scatter(Cloc, [gn], -pos, mask=msub)

    _split = 504
    cp_sa.wait()
    cp_ga.wait()
    plsc.parallel_loop(0, _split, unroll=8)(_accum)
    cp_sb.wait()
    cp_gb.wait()
    plsc.parallel_loop(_split, _NG, unroll=8)(_accum)

    pltpu.sync_copy(Sloc, Ssh.at[sid])
    pltpu.sync_copy(Cloc, Csh.at[sid])
    plsc.subcore_barrier()

    @pl.when(sid < _B // 128)
    def _epilogue():
      col = sid * 128
      pltpu.sync_copy(Ssh.at[:, pl.ds(col, 128)], Stmp)
      pltpu.sync_copy(Csh.at[:, pl.ds(col, 128)], Ctmp)

      for k in range(128 // 16):
        acc_s = Stmp[0, pl.ds(k * 16, 16)]
        acc_c = Ctmp[0, pl.ds(k * 16, 16)]
        for r in range(1, _NS):
          acc_s = acc_s + Stmp[r, pl.ds(k * 16, 16)]
          acc_c = acc_c + Ctmp[r, pl.ds(k * 16, 16)]
        cf = acc_c.astype(jnp.float32)
        d = jnp.float32(3.0) * (cf - jnp.float32(1.0))
        wx = d * _L + jnp.float32(0.5) * (d * _Q) - jnp.float32(0.5) * d
        outv = cf * _WH + cf * wx + jnp.float32(0.5) * (_A2 * acc_s)
        obuf[pl.ds(k * 16, 16)] = outv

      pltpu.sync_copy(obuf, out_hbm.at[pl.ds(col, 128)])

  return _seg_kernel


def kernel(h, x, graph_indices):
    s = _node_sq(h, x).reshape(_N)
    return _build_seg_kernel()(s, graph_indices)
